# baseline (speedup 1.0000x reference)
import functools

import jax
import jax.numpy as jnp
from jax import lax
from jax.experimental import pallas as pl
from jax.experimental.pallas import tpu as pltpu
from jax.experimental.pallas import tpu_sc as plsc

VOCAB = 1000
EMBED = 64
EPAD = 128
B = 4096

_INFO = plsc.get_sparse_core_info()
_NC, _NS = _INFO.num_cores, _INFO.num_subcores
_NW = _NC * _NS
_BPW = B // _NW


_CH = 4
_RCH = _BPW // _CH


def _sc_gather_pair():
  mesh = plsc.VectorSubcoreMesh(core_axis_name="c", subcore_axis_name="s")

  @functools.partial(
      pl.kernel,
      mesh=mesh,
      out_type=[
          jax.ShapeDtypeStruct((B, EPAD), jnp.float32),
          jax.ShapeDtypeStruct((B, EPAD), jnp.float32),
      ],
      scratch_types=[
          pltpu.VMEM((_BPW,), jnp.int32),
          pltpu.VMEM((2, _RCH, EPAD), jnp.float32),
          pltpu.VMEM((_BPW,), jnp.int32),
          pltpu.VMEM((2, _RCH, EPAD), jnp.float32),
          pltpu.SemaphoreType.DMA,
          pltpu.SemaphoreType.DMA,
          pltpu.SemaphoreType.DMA,
          pltpu.SemaphoreType.DMA,
          pltpu.SemaphoreType.DMA,
          pltpu.SemaphoreType.DMA,
      ],
  )
  def k(table_hbm, cen_hbm, ctx_hbm, cen_out, ctx_out,
        idx_a, rows_a, idx_b, rows_b,
        sia, sib, sga, sgb, swa, swb):
    wid = lax.axis_index("s") * _NC + lax.axis_index("c")
    base = wid * _BPW

    ia = pltpu.async_copy(cen_hbm.at[pl.ds(base, _BPW)], idx_a, sia)
    ib = pltpu.async_copy(ctx_hbm.at[pl.ds(base, _BPW)], idx_b, sib)

    chains = [
        dict(idx=idx_a, rows=rows_a, out=cen_out, sg=sga, sw=swa,
             g=[None] * _CH, w=[None] * _CH),
        dict(idx=idx_b, rows=rows_b, out=ctx_out, sg=sgb, sw=swb,
             g=[None] * _CH, w=[None] * _CH),
    ]

    def gather(ch, c, buf):
      return pltpu.async_copy(
          table_hbm.at[ch["idx"].at[pl.ds(c * _RCH, _RCH)]],
          ch["rows"].at[buf], ch["sg"])

    ia.wait()
    chains[0]["g"][0] = gather(chains[0], 0, 0)
    ib.wait()
    chains[1]["g"][0] = gather(chains[1], 0, 0)

    for c in range(_CH):
      buf = c % 2
      for ch in chains:
        if c + 1 < _CH:
          if c >= 1:
            ch["w"][c - 1].wait()
          ch["g"][c + 1] = gather(ch, c + 1, 1 - buf)
        ch["g"][c].wait()
        ch["w"][c] = pltpu.async_copy(
            ch["rows"].at[buf],
            ch["out"].at[pl.ds(base + c * _RCH, _RCH)], ch["sw"])
    for ch in chains:
      ch["w"][_CH - 2].wait()
      ch["w"][_CH - 1].wait()

  return k


_BM = 2048
_BN = 512


_LOG2E = 1.4426950408889634
_LN2 = 0.6931471805599453


def _mm_body(a_ref, b_ref, o_ref):
  a = a_ref[...]
  b = b_ref[...]
  x = lax.dot_general(a, b, (((1,), (1,)), ((), ())),
                      preferred_element_type=jnp.float32)
  t = x * _LOG2E
  ti = lax.bitcast_convert_type(t, jnp.int32)
  neg_abs = lax.bitcast_convert_type(ti | jnp.int32(-2147483648), jnp.float32)
  l = jnp.log2(1.0 + jnp.exp2(neg_abs))
  o_ref[...] = (jnp.minimum(t, 0.0) - l) * _LN2


def _tc_matmul_logsigmoid(cen_emb, ctx_emb):
  grid = (B // _BM, B // _BN)
  return pl.pallas_call(
      _mm_body,
      grid=grid,
      in_specs=[
          pl.BlockSpec((_BM, EPAD), lambda i, j: (i, 0)),
          pl.BlockSpec((_BN, EPAD), lambda i, j: (j, 0)),
      ],
      out_specs=pl.BlockSpec((_BM, _BN), lambda i, j: (i, j)),
      out_shape=jax.ShapeDtypeStruct((B, B), jnp.float32),
      compiler_params=pltpu.CompilerParams(
          dimension_semantics=("parallel", "parallel"),
      ),
  )(cen_emb, ctx_emb)


GP = 1024


def _g_body(t_ref, o_ref):
  t = t_ref[...]
  x = lax.dot_general(t, t, (((1,), (1,)), ((), ())),
                      preferred_element_type=jnp.float32)
  s = x * _LOG2E
  si = lax.bitcast_convert_type(s, jnp.int32)
  neg_abs = lax.bitcast_convert_type(si | jnp.int32(-2147483648), jnp.float32)
  l = jnp.log2(1.0 + jnp.exp2(neg_abs))
  o_ref[...] = (jnp.minimum(s, 0.0) - l) * _LN2


def _tc_gram_logsigmoid(table):
  tpad = jnp.pad(table, ((0, GP - VOCAB), (0, 0)))
  return pl.pallas_call(
      _g_body,
      out_shape=jax.ShapeDtypeStruct((GP, GP), jnp.float32),
  )(tpad)


_R = 8
_NCH = _BPW // _R


def _sc_out_gather():
  mesh = plsc.VectorSubcoreMesh(core_axis_name="c", subcore_axis_name="s")

  @functools.partial(
      pl.kernel,
      mesh=mesh,
      out_type=jax.ShapeDtypeStruct((B, B), jnp.float32),
      compiler_params=pltpu.CompilerParams(needs_layout_passes=False),
      scratch_types=[
          pltpu.VMEM((_BPW,), jnp.int32),
          pltpu.VMEM((B,), jnp.int32),
          pltpu.VMEM((2, _R, GP), jnp.float32),
          pltpu.VMEM((2, _R, B), jnp.float32),
          pltpu.SemaphoreType.DMA,
          pltpu.SemaphoreType.DMA,
          pltpu.SemaphoreType.DMA,
      ],
  )
  def k(g_hbm, cen_hbm, ctx_hbm, out_hbm,
        cen_v, ctx_v, grows, outb, sem_in, sem_g, sem_w):
    wid = lax.axis_index("s") * _NC + lax.axis_index("c")
    base = wid * _BPW
    ia = pltpu.async_copy(cen_hbm.at[pl.ds(base, _BPW)], cen_v, sem_in)
    ib = pltpu.async_copy(ctx_hbm, ctx_v, sem_in)
    ia.wait()
    ib.wait()

    def gather_chunk(c, buf):
      return pltpu.async_copy(
          g_hbm.at[cen_v.at[pl.ds(c * _R, _R)]], grows.at[buf], sem_g)

    cp = gather_chunk(0, 0)
    writes = [None] * _NCH
    for c in range(_NCH):
      buf = c % 2
      cp.wait()
      if c + 1 < _NCH:
        cp = gather_chunk(c + 1, 1 - buf)
      if c >= 2:
        writes[c - 2].wait()
      for r in range(_R):
        bufc = jnp.full((16,), buf, jnp.int32)
        rowc = jnp.full((16,), r, jnp.int32)

        def body(g, carry, _buf=buf, _r=r, _bufc=bufc, _rowc=rowc):
          cvals = ctx_v[pl.ds(g * 16, 16)]
          vals = plsc.load_gather(grows, [_bufc, _rowc, cvals])
          outb[_buf, _r, pl.ds(g * 16, 16)] = vals
          return carry

        lax.fori_loop(0, B // 16, body, 0)
      writes[c] = pltpu.async_copy(
          outb.at[buf], out_hbm.at[pl.ds(base + c * _R, _R)], sem_w)
    writes[_NCH - 2].wait()
    writes[_NCH - 1].wait()

  return k


@jax.jit
def kernel(center_id, context_id, table):
  table_pad = jnp.pad(table, ((0, 0), (0, EPAD - EMBED)))
  cen_emb, ctx_emb = _sc_gather_pair()(
      table_pad, center_id.astype(jnp.int32), context_id.astype(jnp.int32))
  return _tc_matmul_logsigmoid(cen_emb, ctx_emb)


# speedup vs baseline: 1.0510x; 1.0510x over previous
import functools

import jax
import jax.numpy as jnp
from jax import lax
from jax.experimental import pallas as pl
from jax.experimental.pallas import tpu as pltpu
from jax.experimental.pallas import tpu_sc as plsc

VOCAB = 1000
EMBED = 64
EPAD = 128
B = 4096

_INFO = plsc.get_sparse_core_info()
_NC, _NS = _INFO.num_cores, _INFO.num_subcores
_NW = _NC * _NS
_BPW = B // _NW


_CH = 4
_RCH = _BPW // _CH


def _sc_gather_pair():
  mesh = plsc.VectorSubcoreMesh(core_axis_name="c", subcore_axis_name="s")

  @functools.partial(
      pl.kernel,
      mesh=mesh,
      out_type=[
          jax.ShapeDtypeStruct((B, EPAD), jnp.float32),
          jax.ShapeDtypeStruct((B, EPAD), jnp.float32),
      ],
      scratch_types=[
          pltpu.VMEM((_BPW,), jnp.int32),
          pltpu.VMEM((2, _RCH, EPAD), jnp.float32),
          pltpu.VMEM((_BPW,), jnp.int32),
          pltpu.VMEM((2, _RCH, EPAD), jnp.float32),
          pltpu.SemaphoreType.DMA,
          pltpu.SemaphoreType.DMA,
          pltpu.SemaphoreType.DMA,
          pltpu.SemaphoreType.DMA,
          pltpu.SemaphoreType.DMA,
          pltpu.SemaphoreType.DMA,
      ],
  )
  def k(table_hbm, cen_hbm, ctx_hbm, cen_out, ctx_out,
        idx_a, rows_a, idx_b, rows_b,
        sia, sib, sga, sgb, swa, swb):
    wid = lax.axis_index("s") * _NC + lax.axis_index("c")
    base = wid * _BPW

    ia = pltpu.async_copy(cen_hbm.at[pl.ds(base, _BPW)], idx_a, sia)
    ib = pltpu.async_copy(ctx_hbm.at[pl.ds(base, _BPW)], idx_b, sib)

    chains = [
        dict(idx=idx_a, rows=rows_a, out=cen_out, sg=sga, sw=swa,
             g=[None] * _CH, w=[None] * _CH),
        dict(idx=idx_b, rows=rows_b, out=ctx_out, sg=sgb, sw=swb,
             g=[None] * _CH, w=[None] * _CH),
    ]

    def gather(ch, c, buf):
      return pltpu.async_copy(
          table_hbm.at[ch["idx"].at[pl.ds(c * _RCH, _RCH)]],
          ch["rows"].at[buf], ch["sg"])

    ia.wait()
    chains[0]["g"][0] = gather(chains[0], 0, 0)
    ib.wait()
    chains[1]["g"][0] = gather(chains[1], 0, 0)

    for c in range(_CH):
      buf = c % 2
      for ch in chains:
        if c + 1 < _CH:
          if c >= 1:
            ch["w"][c - 1].wait()
          ch["g"][c + 1] = gather(ch, c + 1, 1 - buf)
        ch["g"][c].wait()
        ch["w"][c] = pltpu.async_copy(
            ch["rows"].at[buf],
            ch["out"].at[pl.ds(base + c * _RCH, _RCH)], ch["sw"])
    for ch in chains:
      ch["w"][_CH - 2].wait()
      ch["w"][_CH - 1].wait()

  return k


_BM = 2048
_BN = 1024


_LOG2E = 1.4426950408889634
_LN2 = 0.6931471805599453


def _mm_body(a_ref, b_ref, o_ref):
  a = a_ref[...]
  b = b_ref[...]
  x = lax.dot_general(a, b, (((1,), (1,)), ((), ())),
                      preferred_element_type=jnp.float32)
  t = x * _LOG2E
  ti = lax.bitcast_convert_type(t, jnp.int32)
  neg_abs = lax.bitcast_convert_type(ti | jnp.int32(-2147483648), jnp.float32)
  l = jnp.log2(1.0 + jnp.exp2(neg_abs))
  o_ref[...] = (jnp.minimum(t, 0.0) - l) * _LN2


def _tc_matmul_logsigmoid(cen_emb, ctx_emb):
  grid = (B // _BM, B // _BN)
  return pl.pallas_call(
      _mm_body,
      grid=grid,
      in_specs=[
          pl.BlockSpec((_BM, EPAD), lambda i, j: (i, 0)),
          pl.BlockSpec((_BN, EPAD), lambda i, j: (j, 0)),
      ],
      out_specs=pl.BlockSpec((_BM, _BN), lambda i, j: (i, j)),
      out_shape=jax.ShapeDtypeStruct((B, B), jnp.float32),
      compiler_params=pltpu.CompilerParams(
          dimension_semantics=("parallel", "parallel"),
      ),
  )(cen_emb, ctx_emb)


GP = 1024


def _g_body(t_ref, o_ref):
  t = t_ref[...]
  x = lax.dot_general(t, t, (((1,), (1,)), ((), ())),
                      preferred_element_type=jnp.float32)
  s = x * _LOG2E
  si = lax.bitcast_convert_type(s, jnp.int32)
  neg_abs = lax.bitcast_convert_type(si | jnp.int32(-2147483648), jnp.float32)
  l = jnp.log2(1.0 + jnp.exp2(neg_abs))
  o_ref[...] = (jnp.minimum(s, 0.0) - l) * _LN2


def _tc_gram_logsigmoid(table):
  tpad = jnp.pad(table, ((0, GP - VOCAB), (0, 0)))
  return pl.pallas_call(
      _g_body,
      out_shape=jax.ShapeDtypeStruct((GP, GP), jnp.float32),
  )(tpad)


_R = 8
_NCH = _BPW // _R


def _sc_out_gather():
  mesh = plsc.VectorSubcoreMesh(core_axis_name="c", subcore_axis_name="s")

  @functools.partial(
      pl.kernel,
      mesh=mesh,
      out_type=jax.ShapeDtypeStruct((B, B), jnp.float32),
      compiler_params=pltpu.CompilerParams(needs_layout_passes=False),
      scratch_types=[
          pltpu.VMEM((_BPW,), jnp.int32),
          pltpu.VMEM((B,), jnp.int32),
          pltpu.VMEM((2, _R, GP), jnp.float32),
          pltpu.VMEM((2, _R, B), jnp.float32),
          pltpu.SemaphoreType.DMA,
          pltpu.SemaphoreType.DMA,
          pltpu.SemaphoreType.DMA,
      ],
  )
  def k(g_hbm, cen_hbm, ctx_hbm, out_hbm,
        cen_v, ctx_v, grows, outb, sem_in, sem_g, sem_w):
    wid = lax.axis_index("s") * _NC + lax.axis_index("c")
    base = wid * _BPW
    ia = pltpu.async_copy(cen_hbm.at[pl.ds(base, _BPW)], cen_v, sem_in)
    ib = pltpu.async_copy(ctx_hbm, ctx_v, sem_in)
    ia.wait()
    ib.wait()

    def gather_chunk(c, buf):
      return pltpu.async_copy(
          g_hbm.at[cen_v.at[pl.ds(c * _R, _R)]], grows.at[buf], sem_g)

    cp = gather_chunk(0, 0)
    writes = [None] * _NCH
    for c in range(_NCH):
      buf = c % 2
      cp.wait()
      if c + 1 < _NCH:
        cp = gather_chunk(c + 1, 1 - buf)
      if c >= 2:
        writes[c - 2].wait()
      for r in range(_R):
        bufc = jnp.full((16,), buf, jnp.int32)
        rowc = jnp.full((16,), r, jnp.int32)

        def body(g, carry, _buf=buf, _r=r, _bufc=bufc, _rowc=rowc):
          cvals = ctx_v[pl.ds(g * 16, 16)]
          vals = plsc.load_gather(grows, [_bufc, _rowc, cvals])
          outb[_buf, _r, pl.ds(g * 16, 16)] = vals
          return carry

        lax.fori_loop(0, B // 16, body, 0)
      writes[c] = pltpu.async_copy(
          outb.at[buf], out_hbm.at[pl.ds(base + c * _R, _R)], sem_w)
    writes[_NCH - 2].wait()
    writes[_NCH - 1].wait()

  return k


@jax.jit
def kernel(center_id, context_id, table):
  table_pad = jnp.pad(table, ((0, 0), (0, EPAD - EMBED)))
  cen_emb, ctx_emb = _sc_gather_pair()(
      table_pad, center_id.astype(jnp.int32), context_id.astype(jnp.int32))
  return _tc_matmul_logsigmoid(cen_emb, ctx_emb)


# speedup vs baseline: 1.0583x; 1.0070x over previous
import functools

import jax
import jax.numpy as jnp
from jax import lax
from jax.experimental import pallas as pl
from jax.experimental.pallas import tpu as pltpu
from jax.experimental.pallas import tpu_sc as plsc

VOCAB = 1000
EMBED = 64
EPAD = 128
B = 4096

_INFO = plsc.get_sparse_core_info()
_NC, _NS = _INFO.num_cores, _INFO.num_subcores
_NW = _NC * _NS
_BPW = B // _NW


_CH = 2
_RCH = _BPW // _CH


def _sc_gather_pair():
  mesh = plsc.VectorSubcoreMesh(core_axis_name="c", subcore_axis_name="s")

  @functools.partial(
      pl.kernel,
      mesh=mesh,
      out_type=[
          jax.ShapeDtypeStruct((B, EPAD), jnp.float32),
          jax.ShapeDtypeStruct((B, EPAD), jnp.float32),
      ],
      scratch_types=[
          pltpu.VMEM((_BPW,), jnp.int32),
          pltpu.VMEM((2, _RCH, EPAD), jnp.float32),
          pltpu.VMEM((_BPW,), jnp.int32),
          pltpu.VMEM((2, _RCH, EPAD), jnp.float32),
          pltpu.SemaphoreType.DMA,
          pltpu.SemaphoreType.DMA,
          pltpu.SemaphoreType.DMA,
          pltpu.SemaphoreType.DMA,
          pltpu.SemaphoreType.DMA,
          pltpu.SemaphoreType.DMA,
      ],
  )
  def k(table_hbm, cen_hbm, ctx_hbm, cen_out, ctx_out,
        idx_a, rows_a, idx_b, rows_b,
        sia, sib, sga, sgb, swa, swb):
    wid = lax.axis_index("s") * _NC + lax.axis_index("c")
    base = wid * _BPW

    ia = pltpu.async_copy(cen_hbm.at[pl.ds(base, _BPW)], idx_a, sia)
    ib = pltpu.async_copy(ctx_hbm.at[pl.ds(base, _BPW)], idx_b, sib)

    chains = [
        dict(iw=ia, idx=idx_a, rows=rows_a, out=cen_out, sg=sga, sw=swa,
             g=[None] * _CH, w=[None] * _CH),
        dict(iw=ib, idx=idx_b, rows=rows_b, out=ctx_out, sg=sgb, sw=swb,
             g=[None] * _CH, w=[None] * _CH),
    ]

    def gather(ch, c, buf):
      return pltpu.async_copy(
          table_hbm.at[ch["idx"].at[pl.ds(c * _RCH, _RCH)]],
          ch["rows"].at[buf], ch["sg"])

    for ch in chains:
      ch["iw"].wait()
      ch["g"][0] = gather(ch, 0, 0)

    for c in range(_CH):
      buf = c % 2
      for ch in chains:
        if c + 1 < _CH:
          if c >= 1:
            ch["w"][c - 1].wait()
          ch["g"][c + 1] = gather(ch, c + 1, 1 - buf)
        ch["g"][c].wait()
        ch["w"][c] = pltpu.async_copy(
            ch["rows"].at[buf],
            ch["out"].at[pl.ds(base + c * _RCH, _RCH)], ch["sw"])
    for ch in chains:
      ch["w"][_CH - 2].wait()
      ch["w"][_CH - 1].wait()

  return k


_BM = 2048
_BN = 1024


_LOG2E = 1.4426950408889634
_LN2 = 0.6931471805599453


def _mm_body(a_ref, b_ref, o_ref):
  a = a_ref[...]
  b = b_ref[...]
  x = lax.dot_general(a, b, (((1,), (1,)), ((), ())),
                      preferred_element_type=jnp.float32)
  t = x * _LOG2E
  ti = lax.bitcast_convert_type(t, jnp.int32)
  neg_abs = lax.bitcast_convert_type(ti | jnp.int32(-2147483648), jnp.float32)
  l = jnp.log2(1.0 + jnp.exp2(neg_abs))
  o_ref[...] = (jnp.minimum(t, 0.0) - l) * _LN2


def _tc_matmul_logsigmoid(cen_emb, ctx_emb):
  grid = (B // _BM, B // _BN)
  return pl.pallas_call(
      _mm_body,
      grid=grid,
      in_specs=[
          pl.BlockSpec((_BM, EPAD), lambda i, j: (i, 0)),
          pl.BlockSpec((_BN, EPAD), lambda i, j: (j, 0)),
      ],
      out_specs=pl.BlockSpec((_BM, _BN), lambda i, j: (i, j)),
      out_shape=jax.ShapeDtypeStruct((B, B), jnp.float32),
      compiler_params=pltpu.CompilerParams(
          dimension_semantics=("parallel", "parallel"),
      ),
  )(cen_emb, ctx_emb)


GP = 1024


def _g_body(t_ref, o_ref):
  t = t_ref[...]
  x = lax.dot_general(t, t, (((1,), (1,)), ((), ())),
                      preferred_element_type=jnp.float32)
  s = x * _LOG2E
  si = lax.bitcast_convert_type(s, jnp.int32)
  neg_abs = lax.bitcast_convert_type(si | jnp.int32(-2147483648), jnp.float32)
  l = jnp.log2(1.0 + jnp.exp2(neg_abs))
  o_ref[...] = (jnp.minimum(s, 0.0) - l) * _LN2


def _tc_gram_logsigmoid(table):
  tpad = jnp.pad(table, ((0, GP - VOCAB), (0, 0)))
  return pl.pallas_call(
      _g_body,
      out_shape=jax.ShapeDtypeStruct((GP, GP), jnp.float32),
  )(tpad)


_R = 8
_NCH = _BPW // _R


def _sc_out_gather():
  mesh = plsc.VectorSubcoreMesh(core_axis_name="c", subcore_axis_name="s")

  @functools.partial(
      pl.kernel,
      mesh=mesh,
      out_type=jax.ShapeDtypeStruct((B, B), jnp.float32),
      compiler_params=pltpu.CompilerParams(needs_layout_passes=False),
      scratch_types=[
          pltpu.VMEM((_BPW,), jnp.int32),
          pltpu.VMEM((B,), jnp.int32),
          pltpu.VMEM((2, _R, GP), jnp.float32),
          pltpu.VMEM((2, _R, B), jnp.float32),
          pltpu.SemaphoreType.DMA,
          pltpu.SemaphoreType.DMA,
          pltpu.SemaphoreType.DMA,
      ],
  )
  def k(g_hbm, cen_hbm, ctx_hbm, out_hbm,
        cen_v, ctx_v, grows, outb, sem_in, sem_g, sem_w):
    wid = lax.axis_index("s") * _NC + lax.axis_index("c")
    base = wid * _BPW
    ia = pltpu.async_copy(cen_hbm.at[pl.ds(base, _BPW)], cen_v, sem_in)
    ib = pltpu.async_copy(ctx_hbm, ctx_v, sem_in)
    ia.wait()
    ib.wait()

    def gather_chunk(c, buf):
      return pltpu.async_copy(
          g_hbm.at[cen_v.at[pl.ds(c * _R, _R)]], grows.at[buf], sem_g)

    cp = gather_chunk(0, 0)
    writes = [None] * _NCH
    for c in range(_NCH):
      buf = c % 2
      cp.wait()
      if c + 1 < _NCH:
        cp = gather_chunk(c + 1, 1 - buf)
      if c >= 2:
        writes[c - 2].wait()
      for r in range(_R):
        bufc = jnp.full((16,), buf, jnp.int32)
        rowc = jnp.full((16,), r, jnp.int32)

        def body(g, carry, _buf=buf, _r=r, _bufc=bufc, _rowc=rowc):
          cvals = ctx_v[pl.ds(g * 16, 16)]
          vals = plsc.load_gather(grows, [_bufc, _rowc, cvals])
          outb[_buf, _r, pl.ds(g * 16, 16)] = vals
          return carry

        lax.fori_loop(0, B // 16, body, 0)
      writes[c] = pltpu.async_copy(
          outb.at[buf], out_hbm.at[pl.ds(base + c * _R, _R)], sem_w)
    writes[_NCH - 2].wait()
    writes[_NCH - 1].wait()

  return k


@jax.jit
def kernel(center_id, context_id, table):
  table_pad = jnp.pad(table, ((0, 0), (0, EPAD - EMBED)))
  cen_emb, ctx_emb = _sc_gather_pair()(
      table_pad, center_id.astype(jnp.int32), context_id.astype(jnp.int32))
  return _tc_matmul_logsigmoid(cen_emb, ctx_emb)


# speedup vs baseline: 1.0589x; 1.0006x over previous
import functools

import jax
import jax.numpy as jnp
from jax import lax
from jax.experimental import pallas as pl
from jax.experimental.pallas import tpu as pltpu
from jax.experimental.pallas import tpu_sc as plsc

VOCAB = 1000
EMBED = 64
EPAD = 128
B = 4096

_INFO = plsc.get_sparse_core_info()
_NC, _NS = _INFO.num_cores, _INFO.num_subcores
_NW = _NC * _NS
_BPW = B // _NW


_CH = 2
_RCH = _BPW // _CH


def _sc_gather_pair():
  mesh = plsc.VectorSubcoreMesh(core_axis_name="c", subcore_axis_name="s")

  @functools.partial(
      pl.kernel,
      mesh=mesh,
      out_type=[
          jax.ShapeDtypeStruct((B, EPAD), jnp.float32),
          jax.ShapeDtypeStruct((B, EPAD), jnp.float32),
      ],
      scratch_types=[
          pltpu.VMEM((_BPW,), jnp.int32),
          pltpu.VMEM((2, _RCH, EPAD), jnp.float32),
          pltpu.VMEM((_BPW,), jnp.int32),
          pltpu.VMEM((2, _RCH, EPAD), jnp.float32),
          pltpu.SemaphoreType.DMA,
          pltpu.SemaphoreType.DMA,
          pltpu.SemaphoreType.DMA,
          pltpu.SemaphoreType.DMA,
          pltpu.SemaphoreType.DMA,
          pltpu.SemaphoreType.DMA,
      ],
  )
  def k(table_hbm, cen_hbm, ctx_hbm, cen_out, ctx_out,
        idx_a, rows_a, idx_b, rows_b,
        sia, sib, sga, sgb, swa, swb):
    wid = lax.axis_index("s") * _NC + lax.axis_index("c")
    base = wid * _BPW

    ia = pltpu.async_copy(cen_hbm.at[pl.ds(base, _BPW)], idx_a, sia)
    ib = pltpu.async_copy(ctx_hbm.at[pl.ds(base, _BPW)], idx_b, sib)

    chains = [
        dict(iw=ia, idx=idx_a, rows=rows_a, out=cen_out, sg=sga, sw=swa,
             g=[None] * _CH, w=[None] * _CH),
        dict(iw=ib, idx=idx_b, rows=rows_b, out=ctx_out, sg=sgb, sw=swb,
             g=[None] * _CH, w=[None] * _CH),
    ]

    def gather(ch, c, buf):
      return pltpu.async_copy(
          table_hbm.at[ch["idx"].at[pl.ds(c * _RCH, _RCH)]],
          ch["rows"].at[buf], ch["sg"])

    for ch in chains:
      ch["iw"].wait()
      ch["g"][0] = gather(ch, 0, 0)

    for c in range(_CH):
      buf = c % 2
      for ch in chains:
        if c + 1 < _CH:
          if c >= 1:
            ch["w"][c - 1].wait()
          ch["g"][c + 1] = gather(ch, c + 1, 1 - buf)
        ch["g"][c].wait()
        ch["w"][c] = pltpu.async_copy(
            ch["rows"].at[buf],
            ch["out"].at[pl.ds(base + c * _RCH, _RCH)], ch["sw"])
    for ch in chains:
      ch["w"][_CH - 2].wait()
      ch["w"][_CH - 1].wait()

  return k


_BM = 2048
_BN = 1024


_LOG2E = 1.4426950408889634
_LN2 = 0.6931471805599453


def _mm_body(a_ref, b_ref, o_ref):
  a = a_ref[...]
  b = b_ref[...]
  x = lax.dot_general(a, b, (((1,), (1,)), ((), ())),
                      preferred_element_type=jnp.float32)
  t = x * _LOG2E
  ti = lax.bitcast_convert_type(t, jnp.int32)
  neg_abs = lax.bitcast_convert_type(ti | jnp.int32(-2147483648), jnp.float32)
  l = jnp.log2(1.0 + jnp.exp2(neg_abs))
  o_ref[...] = (jnp.minimum(t, 0.0) - l) * _LN2


def _tc_matmul_logsigmoid(cen_emb, ctx_emb):
  grid = (B // _BM, B // _BN)
  return pl.pallas_call(
      _mm_body,
      grid=grid,
      in_specs=[
          pl.BlockSpec((_BM, EPAD), lambda i, j: (i, 0)),
          pl.BlockSpec((_BN, EPAD), lambda i, j: (j, 0)),
      ],
      out_specs=pl.BlockSpec((_BM, _BN), lambda i, j: (i, j)),
      out_shape=jax.ShapeDtypeStruct((B, B), jnp.float32),
      compiler_params=pltpu.CompilerParams(
          dimension_semantics=("parallel", "parallel"),
      ),
  )(cen_emb, ctx_emb)


@jax.jit
def kernel(center_id, context_id, table):
  table_pad = jnp.pad(table, ((0, 0), (0, EPAD - EMBED)))
  cen_emb, ctx_emb = _sc_gather_pair()(
      table_pad, center_id.astype(jnp.int32), context_id.astype(jnp.int32))
  return _tc_matmul_logsigmoid(cen_emb, ctx_emb)
